# baseline (speedup 1.0000x reference)
import functools

import jax
import jax.numpy as jnp
from jax import lax
from jax.experimental import pallas as pl
from jax.experimental.pallas import tpu as pltpu
from jax.experimental.pallas import tpu_sc as plsc

_NC = 2
_NS = 16
_L = 16
_NW = _NC * _NS
_G = 64
_C = 128


def _rsqrt_newton(x):
    xi = lax.bitcast_convert_type(x, jnp.int32)
    yi = jnp.int32(0x5F3759DF) - (xi >> 1)
    y = lax.bitcast_convert_type(yi, jnp.float32)
    for _ in range(3):
        y = y * (1.5 - 0.5 * x * y * y)
    return y


def _make_deg_dinv_kernel(NPAD, EPAD):
    assert EPAD % (_C * _NW) == 0
    nsch = EPAD // (_C * _NS)
    nwch = EPAD // (_C * _NW)
    nps = NPAD // _NS
    WIN = 4
    mesh = plsc.VectorSubcoreMesh(core_axis_name="c", subcore_axis_name="s")

    @functools.partial(
        pl.kernel, mesh=mesh,
        compiler_params=pltpu.CompilerParams(needs_layout_passes=False),
        out_type=(jax.ShapeDtypeStruct((NPAD,), jnp.float32),
                  jax.ShapeDtypeStruct((EPAD,), jnp.float32)),
        scratch_types=[
            pltpu.VMEM_SHARED((NPAD,), jnp.float32),
            pltpu.VMEM((nsch, _C), jnp.int32),
            pltpu.VMEM((nwch, _C), jnp.int32),
            pltpu.VMEM((_C,), jnp.float32),
            pltpu.VMEM((NPAD,), jnp.float32),
            pltpu.VMEM((_C,), jnp.float32),
            pltpu.VMEM((_C,), jnp.float32),
            pltpu.SemaphoreType.DMA,
            pltpu.SemaphoreType.DMA,
            pltpu.SemaphoreType.DMA,
        ],
    )
    def k(col3_hbm, row3_hbm, zero_hbm, dinv_hbm, dinvrow_hbm,
          deg_sh, col_v, row_v, ones_v, dinv_v, oa_v, ob_v, ws, sa, sb):
        cid = lax.axis_index("c")
        sid = lax.axis_index("s")
        wid = cid * _NS + sid

        def fill_ones(i, _):
            ones_v[pl.ds(i * _L, _L)] = jnp.full((_L,), 1.0, jnp.float32)
            return 0
        lax.fori_loop(0, _C // _L, fill_ones, 0)

        pltpu.sync_copy(col3_hbm.at[sid], col_v)
        pltpu.sync_copy(row3_hbm.at[wid], row_v)
        pltpu.sync_copy(zero_hbm.at[pl.ds(sid * nps, nps)],
                        deg_sh.at[pl.ds(sid * nps, nps)])
        plsc.subcore_barrier()

        def chunk1(i, _):
            pltpu.async_copy(ones_v, deg_sh.at[col_v.at[i]], ws, add=True)

            @pl.when(i >= WIN)
            def _():
                pltpu.make_async_copy(
                    ones_v, deg_sh.at[col_v.at[i]], ws).wait()
            return 0
        lax.fori_loop(0, nsch, chunk1, 0)
        for i in range(min(WIN, nsch)):
            pltpu.make_async_copy(ones_v, deg_sh.at[col_v.at[i]], ws).wait()
        plsc.subcore_barrier()

        pltpu.sync_copy(deg_sh.at[pl.ds(sid * nps, nps)],
                        dinv_v.at[pl.ds(sid * nps, nps)])

        def conv(i, _):
            o = sid * nps + i * _L
            d = dinv_v[pl.ds(o, _L)]
            r = _rsqrt_newton(jnp.maximum(d, 1.0))
            dinv_v[pl.ds(o, _L)] = jnp.where(d > 0, r, 0.0)
            return 0
        lax.fori_loop(0, nps // _L, conv, 0)
        pltpu.sync_copy(dinv_v.at[pl.ds(sid * nps, nps)],
                        deg_sh.at[pl.ds(sid * nps, nps)])
        plsc.subcore_barrier()
        pltpu.sync_copy(deg_sh, dinv_v)

        @pl.when(cid == 0)
        def _():
            pltpu.sync_copy(dinv_v.at[pl.ds(sid * nps, nps)],
                            dinv_hbm.at[pl.ds(sid * nps, nps)])

        base_w = wid * nwch * _C

        def gath(i, obuf):
            for j in range(_C // _L):
                ids = row_v[i, pl.ds(j * _L, _L)]
                obuf[pl.ds(j * _L, _L)] = plsc.load_gather(dinv_v, [ids])

        def store(i, obuf, sem):
            pltpu.async_copy(
                obuf, dinvrow_hbm.at[pl.ds(base_w + i * _C, _C)], sem)

        def swait(i, obuf, sem):
            pltpu.make_async_copy(
                obuf, dinvrow_hbm.at[pl.ds(base_w + i * _C, _C)], sem).wait()

        obufs = ((oa_v, sa), (ob_v, sb))
        npair = nwch // 2

        def chunk3(p, _):
            for b, (obuf, sem) in enumerate(obufs):
                i = 2 * p + b

                @pl.when(i >= 2)
                def _():
                    swait(i - 2, obuf, sem)
                gath(i, obuf)
                store(i, obuf, sem)
            return 0
        lax.fori_loop(0, npair, chunk3, 0)
        if nwch % 2:
            i = nwch - 1
            if i >= 2:
                swait(i - 2, oa_v, sa)
            gath(i, oa_v)
            store(i, oa_v, sa)
            swait(nwch - 1, oa_v, sa)
            if nwch >= 2:
                swait(nwch - 2, ob_v, sb)
        else:
            if nwch >= 2:
                swait(nwch - 2, oa_v, sa)
            if nwch >= 1:
                swait(nwch - 1, ob_v, sb)

    return k


_BLK = 8


def _make_edge_pass_kernel(NPAD, H, EPAD, gather):
    assert EPAD % (_C * _NW * 2 * _BLK) == 0 and NPAD % _NS == 0
    nwch = EPAD // (_C * _NW)
    nblk = nwch // _BLK
    npairs = nblk // 2
    nps = NPAD // _NS
    mesh = plsc.VectorSubcoreMesh(core_axis_name="c", subcore_axis_name="s")

    scratch = [
        pltpu.VMEM_SHARED((NPAD, H), jnp.float32),
        pltpu.VMEM((_C, H), jnp.float32),
        pltpu.VMEM((_C, H), jnp.float32),
        pltpu.VMEM((_BLK, _C), jnp.int32),
        pltpu.VMEM((_BLK, _C), jnp.int32),
        pltpu.VMEM((_BLK, _C), jnp.int32),
        pltpu.VMEM((_BLK, _C), jnp.int32),
        pltpu.SemaphoreType.DMA,
        pltpu.SemaphoreType.DMA,
        pltpu.SemaphoreType.DMA,
        pltpu.SemaphoreType.DMA,
        pltpu.SemaphoreType.DMA,
        pltpu.SemaphoreType.DMA,
        pltpu.SemaphoreType.DMA,
        pltpu.SemaphoreType.DMA,
    ]

    @functools.partial(
        pl.kernel, mesh=mesh,
        compiler_params=pltpu.CompilerParams(needs_layout_passes=False),
        out_type=jax.ShapeDtypeStruct((_NC * NPAD, H), jnp.float32),
        scratch_types=scratch,
    )
    def k(src_hbm, row2_hbm, col2_hbm, init_hbm, out_hbm,
          acc, bufa, bufb, c0, c1, r0, r1, ga, gb, sa, sb, ic0, ic1,
          ir0, ir1):
        cid = lax.axis_index("c")
        sid = lax.axis_index("s")
        wid = cid * _NS + sid
        base_c = wid * nwch

        def idx_copy(b, cset, rset, csem, rsem):
            src = pl.ds((base_c + b * _BLK), _BLK)
            pltpu.async_copy(col2_hbm.at[src, :], cset, csem)
            if gather:
                pltpu.async_copy(row2_hbm.at[src, :], rset, rsem)

        def idx_wait(b, cset, rset, csem, rsem):
            src = pl.ds((base_c + b * _BLK), _BLK)
            pltpu.make_async_copy(col2_hbm.at[src, :], cset, csem).wait()
            if gather:
                pltpu.make_async_copy(row2_hbm.at[src, :], rset, rsem).wait()

        def gstart(b, j, rset, buf, gsem):
            if gather:
                pltpu.async_copy(src_hbm.at[rset.at[j]], buf, gsem)
            else:
                o = (base_c + b * _BLK + j) * _C
                pltpu.async_copy(src_hbm.at[pl.ds(o, _C), :], buf, gsem)

        def gwait(b, j, rset, buf, gsem):
            if gather:
                pltpu.make_async_copy(src_hbm.at[rset.at[j]], buf,
                                      gsem).wait()
            else:
                o = (base_c + b * _BLK + j) * _C
                pltpu.make_async_copy(src_hbm.at[pl.ds(o, _C), :], buf,
                                      gsem).wait()

        def scat(j, cset, buf, ssem):
            pltpu.async_copy(buf, acc.at[cset.at[j]], ssem, add=True)
            pltpu.make_async_copy(buf, acc.at[cset.at[j]], ssem).wait()

        pltpu.sync_copy(init_hbm.at[pl.ds(cid * NPAD + sid * nps, nps), :],
                        acc.at[pl.ds(sid * nps, nps), :])
        idx_copy(0, c0, r0, ic0, ir0)
        idx_copy(1, c1, r1, ic1, ir1)
        idx_wait(0, c0, r0, ic0, ir0)

        bufs = ((bufa, ga, sa), (bufb, gb, sb))
        gstart(0, 0, r0, bufa, ga)
        gstart(0, 1, r0, bufb, gb)
        plsc.subcore_barrier()

        def block(b, cset, rset, csem, rsem, ncset, nrset, ncsem, nrsem,
                  more):
            for j in range(_BLK):
                buf, gsem, ssem = bufs[j % 2]
                gwait(b, j, rset, buf, gsem)
                scat(j, cset, buf, ssem)
                if j < _BLK - 2:
                    gstart(b, j + 2, rset, buf, gsem)
                else:
                    if j == _BLK - 2:
                        if more is None:
                            idx_wait(b + 1, ncset, nrset, ncsem, nrsem)
                        else:
                            @pl.when(more)
                            def _():
                                idx_wait(b + 1, ncset, nrset, ncsem, nrsem)

                    if more is None:
                        gstart(b + 1, j + 2 - _BLK, nrset, buf, gsem)
                    else:
                        @pl.when(more)
                        def _():
                            gstart(b + 1, j + 2 - _BLK, nrset, buf, gsem)

        def pair(bp, _):
            bA = 2 * bp
            more = bp + 1 < npairs
            block(bA, c0, r0, ic0, ir0, c1, r1, ic1, ir1, None)

            @pl.when(more)
            def _():
                idx_copy(bA + 2, c0, r0, ic0, ir0)
            block(bA + 1, c1, r1, ic1, ir1, c0, r0, ic0, ir0, more)

            @pl.when(more)
            def _():
                idx_copy(bA + 3, c1, r1, ic1, ir1)
            return 0
        lax.fori_loop(0, npairs, pair, 0)

        plsc.subcore_barrier()
        pltpu.sync_copy(acc.at[pl.ds(sid * nps, nps), :],
                        out_hbm.at[pl.ds(cid * NPAD + sid * nps, nps), :])

    return k


def _tc_init(xp, Wn, bn, W0, b0, dinv2):
    NPAD, DF = xp.shape
    H = Wn.shape[1]
    BR = 1024

    def body(x_ref, wn_ref, bn_ref, w0_ref, b0_ref, dv_ref, xh_ref, hs_ref):
        xh = jnp.maximum(
            jnp.dot(x_ref[...], wn_ref[...],
                    preferred_element_type=jnp.float32,
                    precision=lax.Precision.DEFAULT) + bn_ref[...], 0.0)
        xh_ref[...] = xh
        hs_ref[...] = dv_ref[...] * (
            jnp.dot(xh, w0_ref[...], preferred_element_type=jnp.float32,
                    precision=lax.Precision.DEFAULT) + b0_ref[...])

    return pl.pallas_call(
        body,
        grid=(NPAD // BR,),
        in_specs=[
            pl.BlockSpec((BR, DF), lambda i: (i, 0)),
            pl.BlockSpec((DF, H), lambda i: (0, 0)),
            pl.BlockSpec((1, H), lambda i: (0, 0)),
            pl.BlockSpec((H, H), lambda i: (0, 0)),
            pl.BlockSpec((1, H), lambda i: (0, 0)),
            pl.BlockSpec((BR, 1), lambda i: (i, 0)),
        ],
        out_specs=[pl.BlockSpec((BR, H), lambda i: (i, 0))] * 2,
        out_shape=[jax.ShapeDtypeStruct((NPAD, H), jnp.float32)] * 2,
    )(xp, Wn, bn, W0, b0, dinv2)


def _tc_edge(ea, We, be, dinv_row2):
    E, DE = ea.shape
    H = We.shape[1]
    BR = 2048

    def body(ea_ref, we_ref, be_ref, dr_ref, out_ref):
        v = jnp.maximum(
            jnp.dot(ea_ref[...], we_ref[...],
                    preferred_element_type=jnp.float32,
                    precision=lax.Precision.DEFAULT) + be_ref[...], 0.0)
        out_ref[...] = dr_ref[...] * v

    return pl.pallas_call(
        body,
        grid=(E // BR,),
        in_specs=[
            pl.BlockSpec((BR, DE), lambda i: (i, 0)),
            pl.BlockSpec((DE, H), lambda i: (0, 0)),
            pl.BlockSpec((1, H), lambda i: (0, 0)),
            pl.BlockSpec((BR, 1), lambda i: (i, 0)),
        ],
        out_specs=pl.BlockSpec((BR, H), lambda i: (i, 0)),
        out_shape=jax.ShapeDtypeStruct((E, H), jnp.float32),
    )(ea, We, be, dinv_row2)


def _tc_layer(r0, r1, xh, dinv2, W, b):
    NPAD, H = xh.shape
    BR = 1024

    def body(r0_ref, r1_ref, xh_ref, dv_ref, w_ref, b_ref, xh1_ref, hs_ref):
        xh1 = dv_ref[...] * (r0_ref[...] + r1_ref[...]) + xh_ref[...]
        xh1_ref[...] = xh1
        hs_ref[...] = dv_ref[...] * (
            jnp.dot(xh1, w_ref[...], preferred_element_type=jnp.float32,
                    precision=lax.Precision.DEFAULT) + b_ref[...])

    blk = pl.BlockSpec((BR, H), lambda i: (i, 0))
    return pl.pallas_call(
        body,
        grid=(NPAD // BR,),
        in_specs=[blk, blk, blk,
                  pl.BlockSpec((BR, 1), lambda i: (i, 0)),
                  pl.BlockSpec((H, H), lambda i: (0, 0)),
                  pl.BlockSpec((1, H), lambda i: (0, 0))],
        out_specs=[blk, blk],
        out_shape=[jax.ShapeDtypeStruct((NPAD, H), jnp.float32)] * 2,
    )(r0, r1, xh, dinv2, W, b)


def _tc_final(r0, r1, xh, dinv2, batch2, Wf0, bf0, Wf1, bf1):
    NPAD, H = xh.shape
    BR = 1024
    grid = NPAD // BR

    def body(r0_ref, r1_ref, xh_ref, dv_ref, b_ref,
             wf0_ref, bf0_ref, wf1_ref, bf1_ref, out_ref, sums, counts):
        i = pl.program_id(0)

        @pl.when(i == 0)
        def _():
            sums[...] = jnp.zeros_like(sums)
            counts[...] = jnp.zeros_like(counts)

        xh3 = dv_ref[...] * (r0_ref[...] + r1_ref[...]) + xh_ref[...]
        gidx = lax.broadcasted_iota(jnp.int32, (1, _G), 1)
        onehot = (b_ref[...] == gidx).astype(jnp.float32)
        sums[...] += lax.dot_general(
            onehot, xh3, (((0,), (0,)), ((), ())),
            preferred_element_type=jnp.float32,
            precision=lax.Precision.HIGHEST)
        counts[...] += lax.dot_general(
            onehot, jnp.ones((onehot.shape[0], 1), jnp.float32),
            (((0,), (0,)), ((), ())), preferred_element_type=jnp.float32,
            precision=lax.Precision.DEFAULT)

        @pl.when(i == grid - 1)
        def _():
            pooled = sums[...] / jnp.maximum(counts[...], 1.0)
            hidden = jnp.maximum(
                jnp.dot(pooled, wf0_ref[...],
                        preferred_element_type=jnp.float32,
                        precision=lax.Precision.DEFAULT) + bf0_ref[...], 0.0)
            out_ref[...] = jnp.dot(
                hidden, wf1_ref[...], preferred_element_type=jnp.float32,
                precision=lax.Precision.DEFAULT) + bf1_ref[...]

    blk = pl.BlockSpec((BR, H), lambda i: (i, 0))
    return pl.pallas_call(
        body,
        grid=(grid,),
        in_specs=[blk, blk, blk,
                  pl.BlockSpec((BR, 1), lambda i: (i, 0)),
                  pl.BlockSpec((BR, 1), lambda i: (i, 0)),
                  pl.BlockSpec((H, H), lambda i: (0, 0)),
                  pl.BlockSpec((1, H), lambda i: (0, 0)),
                  pl.BlockSpec((H, 1), lambda i: (0, 0)),
                  pl.BlockSpec((1, 1), lambda i: (0, 0))],
        out_specs=pl.BlockSpec((_G, 1), lambda i: (0, 0)),
        out_shape=jax.ShapeDtypeStruct((_G, 1), jnp.float32),
        scratch_shapes=[pltpu.VMEM((_G, H), jnp.float32),
                        pltpu.VMEM((_G, 1), jnp.float32)],
    )(r0, r1, xh, dinv2, batch2, Wf0, bf0, Wf1, bf1)


def kernel(x, edge_index, edge_attr, batch, parity_atoms,
           W_node, b_node, W_edge, b_edge,
           W_conv0, b_conv0, W_conv1, b_conv1, W_conv2, b_conv2,
           W_ffn0, b_ffn0, W_ffn1, b_ffn1):
    N, DF = x.shape
    E = edge_index.shape[1]
    H = W_node.shape[1]
    NPAD = ((N + _NS * _NW) // (_NS * _NW)) * (_NS * _NW)
    EQ = _C * _NW * 2 * _BLK
    EPAD = ((E + EQ - 1) // EQ) * EQ

    row = jnp.pad(edge_index[0], (0, EPAD - E), constant_values=N)
    col = jnp.pad(edge_index[1], (0, EPAD - E), constant_values=N)
    eap = jnp.pad(edge_attr, ((0, EPAD - E), (0, 0)))
    nwch = EPAD // (_C * _NW)
    nsch = EPAD // (_C * _NS)
    col_w2 = col.reshape(_NW * nwch, _C)
    row_w2 = row.reshape(_NW * nwch, _C)
    row_w3 = row.reshape(_NW, nwch, _C)
    col_s3 = col.reshape(_NS, nsch, _C)

    xp = jnp.pad(x, ((0, NPAD - N), (0, 0)))
    batchp = jnp.pad(batch, (0, NPAD - N), constant_values=_G)[:, None]
    zeros_n = jnp.zeros((NPAD,), jnp.float32)
    zeros_2nh = jnp.zeros((_NC * NPAD, H), jnp.float32)

    deg_k = _make_deg_dinv_kernel(NPAD, EPAD)
    gath_k = _make_edge_pass_kernel(NPAD, H, EPAD, gather=True)
    lin_k = _make_edge_pass_kernel(NPAD, H, EPAD, gather=False)

    dinv, dinv_row = deg_k(col_s3, row_w3, zeros_n)
    dinv2 = dinv[:, None]

    xh, hs = _tc_init(xp, W_node, b_node.reshape(1, H),
                      W_conv0, b_conv0.reshape(1, H), dinv2)
    ea2 = _tc_edge(eap, W_edge, b_edge.reshape(1, H), dinv_row[:, None])

    S = lin_k(ea2, row_w2, col_w2, zeros_2nh)

    for Wl, bl in ((W_conv1, b_conv1), (W_conv2, b_conv2)):
        r = gath_k(hs, row_w2, col_w2, S)
        xh, hs = _tc_layer(r[:NPAD], r[NPAD:], xh, dinv2, Wl,
                           bl.reshape(1, H))

    r = gath_k(hs, row_w2, col_w2, S)
    out = _tc_final(r[:NPAD], r[NPAD:], xh, dinv2, batchp,
                    W_ffn0, b_ffn0.reshape(1, H),
                    W_ffn1, b_ffn1.reshape(1, 1))
    return out


# speedup vs baseline: 1.8545x; 1.8545x over previous
import functools

import jax
import jax.numpy as jnp
from jax import lax
from jax.experimental import pallas as pl
from jax.experimental.pallas import tpu as pltpu
from jax.experimental.pallas import tpu_sc as plsc

_NC = 2
_NS = 16
_L = 16
_NW = _NC * _NS
_G = 64
_C = 128


def _rsqrt_newton(x):
    xi = lax.bitcast_convert_type(x, jnp.int32)
    yi = jnp.int32(0x5F3759DF) - (xi >> 1)
    y = lax.bitcast_convert_type(yi, jnp.float32)
    for _ in range(3):
        y = y * (1.5 - 0.5 * x * y * y)
    return y


def _make_deg_dinv_kernel(NPAD, EPAD):
    assert EPAD % (_C * _NW) == 0
    nsch = EPAD // (_C * _NS)
    nwch = EPAD // (_C * _NW)
    nps = NPAD // _NS
    WIN = 4
    mesh = plsc.VectorSubcoreMesh(core_axis_name="c", subcore_axis_name="s")

    @functools.partial(
        pl.kernel, mesh=mesh,
        compiler_params=pltpu.CompilerParams(needs_layout_passes=False),
        out_type=(jax.ShapeDtypeStruct((NPAD,), jnp.float32),
                  jax.ShapeDtypeStruct((EPAD,), jnp.float32)),
        scratch_types=[
            pltpu.VMEM_SHARED((NPAD,), jnp.float32),
            pltpu.VMEM((nsch, _C), jnp.int32),
            pltpu.VMEM((nwch, _C), jnp.int32),
            pltpu.VMEM((_C,), jnp.float32),
            pltpu.VMEM((NPAD,), jnp.float32),
            pltpu.VMEM((_C,), jnp.float32),
            pltpu.VMEM((_C,), jnp.float32),
            pltpu.SemaphoreType.DMA,
            pltpu.SemaphoreType.DMA,
            pltpu.SemaphoreType.DMA,
        ],
    )
    def k(col3_hbm, row3_hbm, zero_hbm, dinv_hbm, dinvrow_hbm,
          deg_sh, col_v, row_v, ones_v, dinv_v, oa_v, ob_v, ws, sa, sb):
        cid = lax.axis_index("c")
        sid = lax.axis_index("s")
        wid = cid * _NS + sid

        def fill_ones(i, _):
            ones_v[pl.ds(i * _L, _L)] = jnp.full((_L,), 1.0, jnp.float32)
            return 0
        lax.fori_loop(0, _C // _L, fill_ones, 0)

        pltpu.sync_copy(col3_hbm.at[sid], col_v)
        pltpu.sync_copy(row3_hbm.at[wid], row_v)
        pltpu.sync_copy(zero_hbm.at[pl.ds(sid * nps, nps)],
                        deg_sh.at[pl.ds(sid * nps, nps)])
        plsc.subcore_barrier()

        def chunk1(i, _):
            pltpu.async_copy(ones_v, deg_sh.at[col_v.at[i]], ws, add=True)

            @pl.when(i >= WIN)
            def _():
                pltpu.make_async_copy(
                    ones_v, deg_sh.at[col_v.at[i]], ws).wait()
            return 0
        lax.fori_loop(0, nsch, chunk1, 0)
        for i in range(min(WIN, nsch)):
            pltpu.make_async_copy(ones_v, deg_sh.at[col_v.at[i]], ws).wait()
        plsc.subcore_barrier()

        pltpu.sync_copy(deg_sh.at[pl.ds(sid * nps, nps)],
                        dinv_v.at[pl.ds(sid * nps, nps)])

        def conv(i, _):
            o = sid * nps + i * _L
            d = dinv_v[pl.ds(o, _L)]
            r = _rsqrt_newton(jnp.maximum(d, 1.0))
            dinv_v[pl.ds(o, _L)] = jnp.where(d > 0, r, 0.0)
            return 0
        lax.fori_loop(0, nps // _L, conv, 0)
        pltpu.sync_copy(dinv_v.at[pl.ds(sid * nps, nps)],
                        deg_sh.at[pl.ds(sid * nps, nps)])
        plsc.subcore_barrier()
        pltpu.sync_copy(deg_sh, dinv_v)

        @pl.when(cid == 0)
        def _():
            pltpu.sync_copy(dinv_v.at[pl.ds(sid * nps, nps)],
                            dinv_hbm.at[pl.ds(sid * nps, nps)])

        base_w = wid * nwch * _C

        def gath(i, obuf):
            for j in range(_C // _L):
                ids = row_v[i, pl.ds(j * _L, _L)]
                obuf[pl.ds(j * _L, _L)] = plsc.load_gather(dinv_v, [ids])

        def store(i, obuf, sem):
            pltpu.async_copy(
                obuf, dinvrow_hbm.at[pl.ds(base_w + i * _C, _C)], sem)

        def swait(i, obuf, sem):
            pltpu.make_async_copy(
                obuf, dinvrow_hbm.at[pl.ds(base_w + i * _C, _C)], sem).wait()

        obufs = ((oa_v, sa), (ob_v, sb))
        npair = nwch // 2

        def chunk3(p, _):
            for b, (obuf, sem) in enumerate(obufs):
                i = 2 * p + b

                @pl.when(i >= 2)
                def _():
                    swait(i - 2, obuf, sem)
                gath(i, obuf)
                store(i, obuf, sem)
            return 0
        lax.fori_loop(0, npair, chunk3, 0)
        if nwch % 2:
            i = nwch - 1
            if i >= 2:
                swait(i - 2, oa_v, sa)
            gath(i, oa_v)
            store(i, oa_v, sa)
            swait(nwch - 1, oa_v, sa)
            if nwch >= 2:
                swait(nwch - 2, ob_v, sb)
        else:
            if nwch >= 2:
                swait(nwch - 2, oa_v, sa)
            if nwch >= 1:
                swait(nwch - 1, ob_v, sb)

    return k


_BLK = 8


def _make_edge_pass_kernel(NPAD, H, EPAD, gather):
    assert EPAD % (_C * _NW * 2 * _BLK) == 0 and NPAD % _NS == 0
    nwch = EPAD // (_C * _NW)
    nblk = nwch // _BLK
    npairs = nblk // 2
    nps = NPAD // _NS
    mesh = plsc.VectorSubcoreMesh(core_axis_name="c", subcore_axis_name="s")

    scratch = [
        pltpu.VMEM_SHARED((NPAD, H), jnp.float32),
        pltpu.VMEM((_C, H), jnp.float32),
        pltpu.VMEM((_C, H), jnp.float32),
        pltpu.VMEM((_BLK, _C), jnp.int32),
        pltpu.VMEM((_BLK, _C), jnp.int32),
        pltpu.VMEM((_BLK, _C), jnp.int32),
        pltpu.VMEM((_BLK, _C), jnp.int32),
        pltpu.SemaphoreType.DMA,
        pltpu.SemaphoreType.DMA,
        pltpu.SemaphoreType.DMA,
        pltpu.SemaphoreType.DMA,
        pltpu.SemaphoreType.DMA,
        pltpu.SemaphoreType.DMA,
        pltpu.SemaphoreType.DMA,
        pltpu.SemaphoreType.DMA,
    ]

    @functools.partial(
        pl.kernel, mesh=mesh,
        compiler_params=pltpu.CompilerParams(needs_layout_passes=False),
        out_type=jax.ShapeDtypeStruct((_NC * NPAD, H), jnp.float32),
        scratch_types=scratch,
    )
    def k(src_hbm, row2_hbm, col2_hbm, init_hbm, out_hbm,
          acc, bufa, bufb, c0, c1, r0, r1, ga, gb, sa, sb, ic0, ic1,
          ir0, ir1):
        cid = lax.axis_index("c")
        sid = lax.axis_index("s")
        wid = cid * _NS + sid
        base_c = wid * nwch

        def idx_copy(b, cset, rset, csem, rsem):
            src = pl.ds((base_c + b * _BLK), _BLK)
            pltpu.async_copy(col2_hbm.at[src, :], cset, csem)
            if gather:
                pltpu.async_copy(row2_hbm.at[src, :], rset, rsem)

        def idx_wait(b, cset, rset, csem, rsem):
            src = pl.ds((base_c + b * _BLK), _BLK)
            pltpu.make_async_copy(col2_hbm.at[src, :], cset, csem).wait()
            if gather:
                pltpu.make_async_copy(row2_hbm.at[src, :], rset, rsem).wait()

        def gstart(b, j, rset, buf, gsem):
            if gather:
                pltpu.async_copy(src_hbm.at[rset.at[j]], buf, gsem)
            else:
                o = (base_c + b * _BLK + j) * _C
                pltpu.async_copy(src_hbm.at[pl.ds(o, _C), :], buf, gsem)

        def gwait(b, j, rset, buf, gsem):
            if gather:
                pltpu.make_async_copy(src_hbm.at[rset.at[j]], buf,
                                      gsem).wait()
            else:
                o = (base_c + b * _BLK + j) * _C
                pltpu.make_async_copy(src_hbm.at[pl.ds(o, _C), :], buf,
                                      gsem).wait()

        def scat(j, cset, buf, ssem):
            pltpu.async_copy(buf, acc.at[cset.at[j]], ssem, add=True)
            pltpu.make_async_copy(buf, acc.at[cset.at[j]], ssem).wait()

        pltpu.sync_copy(init_hbm.at[pl.ds(cid * NPAD + sid * nps, nps), :],
                        acc.at[pl.ds(sid * nps, nps), :])
        idx_copy(0, c0, r0, ic0, ir0)
        idx_copy(1, c1, r1, ic1, ir1)
        idx_wait(0, c0, r0, ic0, ir0)

        bufs = ((bufa, ga, sa), (bufb, gb, sb))
        gstart(0, 0, r0, bufa, ga)
        gstart(0, 1, r0, bufb, gb)
        plsc.subcore_barrier()

        def block(b, cset, rset, csem, rsem, ncset, nrset, ncsem, nrsem,
                  more):
            for j in range(_BLK):
                buf, gsem, ssem = bufs[j % 2]
                gwait(b, j, rset, buf, gsem)
                scat(j, cset, buf, ssem)
                if j < _BLK - 2:
                    gstart(b, j + 2, rset, buf, gsem)
                else:
                    if j == _BLK - 2:
                        if more is None:
                            idx_wait(b + 1, ncset, nrset, ncsem, nrsem)
                        else:
                            @pl.when(more)
                            def _():
                                idx_wait(b + 1, ncset, nrset, ncsem, nrsem)

                    if more is None:
                        gstart(b + 1, j + 2 - _BLK, nrset, buf, gsem)
                    else:
                        @pl.when(more)
                        def _():
                            gstart(b + 1, j + 2 - _BLK, nrset, buf, gsem)

        def pair(bp, _):
            bA = 2 * bp
            more = bp + 1 < npairs
            block(bA, c0, r0, ic0, ir0, c1, r1, ic1, ir1, None)

            @pl.when(more)
            def _():
                idx_copy(bA + 2, c0, r0, ic0, ir0)
            block(bA + 1, c1, r1, ic1, ir1, c0, r0, ic0, ir0, more)

            @pl.when(more)
            def _():
                idx_copy(bA + 3, c1, r1, ic1, ir1)
            return 0
        lax.fori_loop(0, npairs, pair, 0)

        plsc.subcore_barrier()
        pltpu.sync_copy(acc.at[pl.ds(sid * nps, nps), :],
                        out_hbm.at[pl.ds(cid * NPAD + sid * nps, nps), :])

    return k


def _tc_init(xp, Wn, bn, W0, b0, dinv2):
    NPAD, DF = xp.shape
    H = Wn.shape[1]
    BR = 1024

    def body(x_ref, wn_ref, bn_ref, w0_ref, b0_ref, dv_ref, xh_ref, hs_ref):
        xh = jnp.maximum(
            jnp.dot(x_ref[...], wn_ref[...],
                    preferred_element_type=jnp.float32,
                    precision=lax.Precision.DEFAULT) + bn_ref[...], 0.0)
        xh_ref[...] = xh
        hs_ref[...] = dv_ref[...] * (
            jnp.dot(xh, w0_ref[...], preferred_element_type=jnp.float32,
                    precision=lax.Precision.DEFAULT) + b0_ref[...])

    return pl.pallas_call(
        body,
        grid=(NPAD // BR,),
        in_specs=[
            pl.BlockSpec((BR, DF), lambda i: (i, 0)),
            pl.BlockSpec((DF, H), lambda i: (0, 0)),
            pl.BlockSpec((1, H), lambda i: (0, 0)),
            pl.BlockSpec((H, H), lambda i: (0, 0)),
            pl.BlockSpec((1, H), lambda i: (0, 0)),
            pl.BlockSpec((BR, 1), lambda i: (i, 0)),
        ],
        out_specs=[pl.BlockSpec((BR, H), lambda i: (i, 0))] * 2,
        out_shape=[jax.ShapeDtypeStruct((NPAD, H), jnp.float32)] * 2,
    )(xp, Wn, bn, W0, b0, dinv2)


def _tc_edge(ea, We, be, dinv_row2):
    E, DE = ea.shape
    H = We.shape[1]
    BR = 2048

    def body(ea_ref, we_ref, be_ref, dr_ref, out_ref):
        v = jnp.maximum(
            jnp.dot(ea_ref[...], we_ref[...],
                    preferred_element_type=jnp.float32,
                    precision=lax.Precision.DEFAULT) + be_ref[...], 0.0)
        out_ref[...] = dr_ref[...] * v

    return pl.pallas_call(
        body,
        grid=(E // BR,),
        in_specs=[
            pl.BlockSpec((BR, DE), lambda i: (i, 0)),
            pl.BlockSpec((DE, H), lambda i: (0, 0)),
            pl.BlockSpec((1, H), lambda i: (0, 0)),
            pl.BlockSpec((BR, 1), lambda i: (i, 0)),
        ],
        out_specs=pl.BlockSpec((BR, H), lambda i: (i, 0)),
        out_shape=jax.ShapeDtypeStruct((E, H), jnp.float32),
    )(ea, We, be, dinv_row2)


def _tc_layer(r0, r1, xh, dinv2, W, b):
    NPAD, H = xh.shape
    BR = 1024

    def body(r0_ref, r1_ref, xh_ref, dv_ref, w_ref, b_ref, xh1_ref, hs_ref):
        xh1 = dv_ref[...] * (r0_ref[...] + r1_ref[...]) + xh_ref[...]
        xh1_ref[...] = xh1
        hs_ref[...] = dv_ref[...] * (
            jnp.dot(xh1, w_ref[...], preferred_element_type=jnp.float32,
                    precision=lax.Precision.DEFAULT) + b_ref[...])

    blk = pl.BlockSpec((BR, H), lambda i: (i, 0))
    return pl.pallas_call(
        body,
        grid=(NPAD // BR,),
        in_specs=[blk, blk, blk,
                  pl.BlockSpec((BR, 1), lambda i: (i, 0)),
                  pl.BlockSpec((H, H), lambda i: (0, 0)),
                  pl.BlockSpec((1, H), lambda i: (0, 0))],
        out_specs=[blk, blk],
        out_shape=[jax.ShapeDtypeStruct((NPAD, H), jnp.float32)] * 2,
    )(r0, r1, xh, dinv2, W, b)


def _tc_final(r0, r1, xh, dinv2, batch2, Wf0, bf0, Wf1, bf1):
    NPAD, H = xh.shape
    BR = 1024
    grid = NPAD // BR

    def body(r0_ref, r1_ref, xh_ref, dv_ref, b_ref,
             wf0_ref, bf0_ref, wf1_ref, bf1_ref, out_ref, sums, counts):
        i = pl.program_id(0)

        @pl.when(i == 0)
        def _():
            sums[...] = jnp.zeros_like(sums)
            counts[...] = jnp.zeros_like(counts)

        xh3 = dv_ref[...] * (r0_ref[...] + r1_ref[...]) + xh_ref[...]
        gidx = lax.broadcasted_iota(jnp.int32, (1, _G), 1)
        onehot = (b_ref[...] == gidx).astype(jnp.float32)
        sums[...] += lax.dot_general(
            onehot, xh3, (((0,), (0,)), ((), ())),
            preferred_element_type=jnp.float32,
            precision=lax.Precision.HIGHEST)
        counts[...] += lax.dot_general(
            onehot, jnp.ones((onehot.shape[0], 1), jnp.float32),
            (((0,), (0,)), ((), ())), preferred_element_type=jnp.float32,
            precision=lax.Precision.DEFAULT)

        @pl.when(i == grid - 1)
        def _():
            pooled = sums[...] / jnp.maximum(counts[...], 1.0)
            hidden = jnp.maximum(
                jnp.dot(pooled, wf0_ref[...],
                        preferred_element_type=jnp.float32,
                        precision=lax.Precision.DEFAULT) + bf0_ref[...], 0.0)
            out_ref[...] = jnp.dot(
                hidden, wf1_ref[...], preferred_element_type=jnp.float32,
                precision=lax.Precision.DEFAULT) + bf1_ref[...]

    blk = pl.BlockSpec((BR, H), lambda i: (i, 0))
    return pl.pallas_call(
        body,
        grid=(grid,),
        in_specs=[blk, blk, blk,
                  pl.BlockSpec((BR, 1), lambda i: (i, 0)),
                  pl.BlockSpec((BR, 1), lambda i: (i, 0)),
                  pl.BlockSpec((H, H), lambda i: (0, 0)),
                  pl.BlockSpec((1, H), lambda i: (0, 0)),
                  pl.BlockSpec((H, 1), lambda i: (0, 0)),
                  pl.BlockSpec((1, 1), lambda i: (0, 0))],
        out_specs=pl.BlockSpec((_G, 1), lambda i: (0, 0)),
        out_shape=jax.ShapeDtypeStruct((_G, 1), jnp.float32),
        scratch_shapes=[pltpu.VMEM((_G, H), jnp.float32),
                        pltpu.VMEM((_G, 1), jnp.float32)],
    )(r0, r1, xh, dinv2, batch2, Wf0, bf0, Wf1, bf1)


def kernel(x, edge_index, edge_attr, batch, parity_atoms,
           W_node, b_node, W_edge, b_edge,
           W_conv0, b_conv0, W_conv1, b_conv1, W_conv2, b_conv2,
           W_ffn0, b_ffn0, W_ffn1, b_ffn1):
    N, DF = x.shape
    E = edge_index.shape[1]
    H = W_node.shape[1]
    NPAD = ((N + _NS * _NW) // (_NS * _NW)) * (_NS * _NW)
    EQ = _C * _NW * 2 * _BLK
    EPAD = ((E + EQ - 1) // EQ) * EQ

    padidx = (jnp.arange(EPAD - E, dtype=jnp.int32) % (NPAD - N)) + N
    row = jnp.concatenate([edge_index[0], padidx])
    col = jnp.concatenate([edge_index[1], padidx])
    eap = jnp.pad(edge_attr, ((0, EPAD - E), (0, 0)))
    nwch = EPAD // (_C * _NW)
    nsch = EPAD // (_C * _NS)
    col_w2 = col.reshape(_NW * nwch, _C)
    row_w2 = row.reshape(_NW * nwch, _C)
    row_w3 = row.reshape(_NW, nwch, _C)
    col_s3 = col.reshape(_NS, nsch, _C)

    xp = jnp.pad(x, ((0, NPAD - N), (0, 0)))
    batchp = jnp.pad(batch, (0, NPAD - N), constant_values=_G)[:, None]
    zeros_n = jnp.zeros((NPAD,), jnp.float32)
    zeros_2nh = jnp.zeros((_NC * NPAD, H), jnp.float32)

    deg_k = _make_deg_dinv_kernel(NPAD, EPAD)
    gath_k = _make_edge_pass_kernel(NPAD, H, EPAD, gather=True)
    lin_k = _make_edge_pass_kernel(NPAD, H, EPAD, gather=False)

    dinv, dinv_row = deg_k(col_s3, row_w3, zeros_n)
    dinv2 = dinv[:, None]

    xh, hs = _tc_init(xp, W_node, b_node.reshape(1, H),
                      W_conv0, b_conv0.reshape(1, H), dinv2)
    ea2 = _tc_edge(eap, W_edge, b_edge.reshape(1, H), dinv_row[:, None])

    S = lin_k(ea2, row_w2, col_w2, zeros_2nh)

    for Wl, bl in ((W_conv1, b_conv1), (W_conv2, b_conv2)):
        r = gath_k(hs, row_w2, col_w2, S)
        xh, hs = _tc_layer(r[:NPAD], r[NPAD:], xh, dinv2, Wl,
                           bl.reshape(1, H))

    r = gath_k(hs, row_w2, col_w2, S)
    out = _tc_final(r[:NPAD], r[NPAD:], xh, dinv2, batchp,
                    W_ffn0, b_ffn0.reshape(1, H),
                    W_ffn1, b_ffn1.reshape(1, 1))
    return out


# speedup vs baseline: 1.8548x; 1.0002x over previous
import functools

import jax
import jax.numpy as jnp
from jax import lax
from jax.experimental import pallas as pl
from jax.experimental.pallas import tpu as pltpu
from jax.experimental.pallas import tpu_sc as plsc

_NC = 2
_NS = 16
_L = 16
_NW = _NC * _NS
_G = 64
_C = 128


def _rsqrt_newton(x):
    xi = lax.bitcast_convert_type(x, jnp.int32)
    yi = jnp.int32(0x5F3759DF) - (xi >> 1)
    y = lax.bitcast_convert_type(yi, jnp.float32)
    for _ in range(3):
        y = y * (1.5 - 0.5 * x * y * y)
    return y


def _make_deg_dinv_kernel(NPAD, EPAD):
    assert EPAD % (_C * _NW) == 0
    nsch = EPAD // (_C * _NS)
    nwch = EPAD // (_C * _NW)
    nps = NPAD // _NS
    WIN = 4
    mesh = plsc.VectorSubcoreMesh(core_axis_name="c", subcore_axis_name="s")

    @functools.partial(
        pl.kernel, mesh=mesh,
        compiler_params=pltpu.CompilerParams(needs_layout_passes=False),
        out_type=(jax.ShapeDtypeStruct((NPAD,), jnp.float32),
                  jax.ShapeDtypeStruct((EPAD,), jnp.float32)),
        scratch_types=[
            pltpu.VMEM_SHARED((NPAD,), jnp.float32),
            pltpu.VMEM((nsch, _C), jnp.int32),
            pltpu.VMEM((nwch, _C), jnp.int32),
            pltpu.VMEM((_C,), jnp.float32),
            pltpu.VMEM((NPAD,), jnp.float32),
            pltpu.VMEM((_C,), jnp.float32),
            pltpu.VMEM((_C,), jnp.float32),
            pltpu.SemaphoreType.DMA,
            pltpu.SemaphoreType.DMA,
            pltpu.SemaphoreType.DMA,
        ],
    )
    def k(col3_hbm, row3_hbm, zero_hbm, dinv_hbm, dinvrow_hbm,
          deg_sh, col_v, row_v, ones_v, dinv_v, oa_v, ob_v, ws, sa, sb):
        cid = lax.axis_index("c")
        sid = lax.axis_index("s")
        wid = cid * _NS + sid

        def fill_ones(i, _):
            ones_v[pl.ds(i * _L, _L)] = jnp.full((_L,), 1.0, jnp.float32)
            return 0
        lax.fori_loop(0, _C // _L, fill_ones, 0)

        pltpu.sync_copy(col3_hbm.at[sid], col_v)
        pltpu.sync_copy(row3_hbm.at[wid], row_v)
        pltpu.sync_copy(zero_hbm.at[pl.ds(sid * nps, nps)],
                        deg_sh.at[pl.ds(sid * nps, nps)])
        plsc.subcore_barrier()

        def chunk1(i, _):
            pltpu.async_copy(ones_v, deg_sh.at[col_v.at[i]], ws, add=True)

            @pl.when(i >= WIN)
            def _():
                pltpu.make_async_copy(
                    ones_v, deg_sh.at[col_v.at[i]], ws).wait()
            return 0
        lax.fori_loop(0, nsch, chunk1, 0)
        for i in range(min(WIN, nsch)):
            pltpu.make_async_copy(ones_v, deg_sh.at[col_v.at[i]], ws).wait()
        plsc.subcore_barrier()

        pltpu.sync_copy(deg_sh.at[pl.ds(sid * nps, nps)],
                        dinv_v.at[pl.ds(sid * nps, nps)])

        def conv(i, _):
            o = sid * nps + i * _L
            d = dinv_v[pl.ds(o, _L)]
            r = _rsqrt_newton(jnp.maximum(d, 1.0))
            dinv_v[pl.ds(o, _L)] = jnp.where(d > 0, r, 0.0)
            return 0
        lax.fori_loop(0, nps // _L, conv, 0)
        pltpu.sync_copy(dinv_v.at[pl.ds(sid * nps, nps)],
                        deg_sh.at[pl.ds(sid * nps, nps)])
        plsc.subcore_barrier()
        pltpu.sync_copy(deg_sh, dinv_v)

        @pl.when(cid == 0)
        def _():
            pltpu.sync_copy(dinv_v.at[pl.ds(sid * nps, nps)],
                            dinv_hbm.at[pl.ds(sid * nps, nps)])

        base_w = wid * nwch * _C

        def gath(i, obuf):
            for j in range(_C // _L):
                ids = row_v[i, pl.ds(j * _L, _L)]
                obuf[pl.ds(j * _L, _L)] = plsc.load_gather(dinv_v, [ids])

        def store(i, obuf, sem):
            pltpu.async_copy(
                obuf, dinvrow_hbm.at[pl.ds(base_w + i * _C, _C)], sem)

        def swait(i, obuf, sem):
            pltpu.make_async_copy(
                obuf, dinvrow_hbm.at[pl.ds(base_w + i * _C, _C)], sem).wait()

        obufs = ((oa_v, sa), (ob_v, sb))
        npair = nwch // 2

        def chunk3(p, _):
            for b, (obuf, sem) in enumerate(obufs):
                i = 2 * p + b

                @pl.when(i >= 2)
                def _():
                    swait(i - 2, obuf, sem)
                gath(i, obuf)
                store(i, obuf, sem)
            return 0
        lax.fori_loop(0, npair, chunk3, 0)
        if nwch % 2:
            i = nwch - 1
            if i >= 2:
                swait(i - 2, oa_v, sa)
            gath(i, oa_v)
            store(i, oa_v, sa)
            swait(nwch - 1, oa_v, sa)
            if nwch >= 2:
                swait(nwch - 2, ob_v, sb)
        else:
            if nwch >= 2:
                swait(nwch - 2, oa_v, sa)
            if nwch >= 1:
                swait(nwch - 1, ob_v, sb)

    return k


_BLK = 8


def _make_edge_pass_kernel(NPAD, H, EPAD, gather, zero_init=False):
    assert EPAD % (_C * _NW * 2 * _BLK) == 0 and NPAD % (_NS * _C) == 0
    nwch = EPAD // (_C * _NW)
    nblk = nwch // _BLK
    npairs = nblk // 2
    nps = NPAD // _NS
    mesh = plsc.VectorSubcoreMesh(core_axis_name="c", subcore_axis_name="s")

    scratch = [
        pltpu.VMEM_SHARED((NPAD, H), jnp.float32),
        pltpu.VMEM((_C, H), jnp.float32),
        pltpu.VMEM((_C, H), jnp.float32),
        pltpu.VMEM((_BLK, _C), jnp.int32),
        pltpu.VMEM((_BLK, _C), jnp.int32),
        pltpu.VMEM((_BLK, _C), jnp.int32),
        pltpu.VMEM((_BLK, _C), jnp.int32),
        pltpu.SemaphoreType.DMA,
        pltpu.SemaphoreType.DMA,
        pltpu.SemaphoreType.DMA,
        pltpu.SemaphoreType.DMA,
        pltpu.SemaphoreType.DMA,
        pltpu.SemaphoreType.DMA,
        pltpu.SemaphoreType.DMA,
        pltpu.SemaphoreType.DMA,
    ]

    def body(src_hbm, row2_hbm, col2_hbm, init_hbm, out_hbm,
             acc, bufa, bufb, c0, c1, r0, r1, ga, gb, sa, sb, ic0, ic1,
             ir0, ir1):
        cid = lax.axis_index("c")
        sid = lax.axis_index("s")
        wid = cid * _NS + sid
        base_c = wid * nwch

        def idx_copy(b, cset, rset, csem, rsem):
            src = pl.ds((base_c + b * _BLK), _BLK)
            pltpu.async_copy(col2_hbm.at[src, :], cset, csem)
            if gather:
                pltpu.async_copy(row2_hbm.at[src, :], rset, rsem)

        def idx_wait(b, cset, rset, csem, rsem):
            src = pl.ds((base_c + b * _BLK), _BLK)
            pltpu.make_async_copy(col2_hbm.at[src, :], cset, csem).wait()
            if gather:
                pltpu.make_async_copy(row2_hbm.at[src, :], rset, rsem).wait()

        def gstart(b, j, rset, buf, gsem):
            if gather:
                pltpu.async_copy(src_hbm.at[rset.at[j]], buf, gsem)
            else:
                o = (base_c + b * _BLK + j) * _C
                pltpu.async_copy(src_hbm.at[pl.ds(o, _C), :], buf, gsem)

        def gwait(b, j, rset, buf, gsem):
            if gather:
                pltpu.make_async_copy(src_hbm.at[rset.at[j]], buf,
                                      gsem).wait()
            else:
                o = (base_c + b * _BLK + j) * _C
                pltpu.make_async_copy(src_hbm.at[pl.ds(o, _C), :], buf,
                                      gsem).wait()

        def scat(j, cset, buf, ssem):
            pltpu.async_copy(buf, acc.at[cset.at[j]], ssem, add=True)
            pltpu.make_async_copy(buf, acc.at[cset.at[j]], ssem).wait()

        if zero_init:
            def zrow(rr, _):
                for cc in range(H // _L):
                    bufa[rr, pl.ds(cc * _L, _L)] = jnp.zeros((_L,),
                                                             jnp.float32)
                return 0
            lax.fori_loop(0, _C, zrow, 0)
            for t in range(nps // _C):
                pltpu.async_copy(
                    bufa, acc.at[pl.ds(sid * nps + t * _C, _C), :], ga)
            idx_copy(0, c0, r0, ic0, ir0)
            idx_copy(1, c1, r1, ic1, ir1)
            for t in range(nps // _C):
                pltpu.make_async_copy(
                    bufa, acc.at[pl.ds(sid * nps + t * _C, _C), :],
                    ga).wait()
        else:
            pltpu.sync_copy(
                init_hbm.at[pl.ds(cid * NPAD + sid * nps, nps), :],
                acc.at[pl.ds(sid * nps, nps), :])
            idx_copy(0, c0, r0, ic0, ir0)
            idx_copy(1, c1, r1, ic1, ir1)
        idx_wait(0, c0, r0, ic0, ir0)

        bufs = ((bufa, ga, sa), (bufb, gb, sb))
        gstart(0, 0, r0, bufa, ga)
        gstart(0, 1, r0, bufb, gb)
        plsc.subcore_barrier()

        def block(b, cset, rset, csem, rsem, ncset, nrset, ncsem, nrsem,
                  more):
            for j in range(_BLK):
                buf, gsem, ssem = bufs[j % 2]
                gwait(b, j, rset, buf, gsem)
                scat(j, cset, buf, ssem)
                if j < _BLK - 2:
                    gstart(b, j + 2, rset, buf, gsem)
                else:
                    if j == _BLK - 2:
                        if more is None:
                            idx_wait(b + 1, ncset, nrset, ncsem, nrsem)
                        else:
                            @pl.when(more)
                            def _():
                                idx_wait(b + 1, ncset, nrset, ncsem, nrsem)

                    if more is None:
                        gstart(b + 1, j + 2 - _BLK, nrset, buf, gsem)
                    else:
                        @pl.when(more)
                        def _():
                            gstart(b + 1, j + 2 - _BLK, nrset, buf, gsem)

        def pair(bp, _):
            bA = 2 * bp
            more = bp + 1 < npairs
            block(bA, c0, r0, ic0, ir0, c1, r1, ic1, ir1, None)

            @pl.when(more)
            def _():
                idx_copy(bA + 2, c0, r0, ic0, ir0)
            block(bA + 1, c1, r1, ic1, ir1, c0, r0, ic0, ir0, more)

            @pl.when(more)
            def _():
                idx_copy(bA + 3, c1, r1, ic1, ir1)
            return 0
        lax.fori_loop(0, npairs, pair, 0)

        plsc.subcore_barrier()
        pltpu.sync_copy(acc.at[pl.ds(sid * nps, nps), :],
                        out_hbm.at[pl.ds(cid * NPAD + sid * nps, nps), :])

    kern = functools.partial(
        pl.kernel, mesh=mesh,
        compiler_params=pltpu.CompilerParams(needs_layout_passes=False),
        out_type=jax.ShapeDtypeStruct((_NC * NPAD, H), jnp.float32),
        scratch_types=scratch,
    )
    if zero_init:
        @kern
        def k(src_hbm, row2_hbm, col2_hbm, out_hbm,
              acc, bufa, bufb, c0, c1, r0, r1, ga, gb, sa, sb, ic0, ic1,
              ir0, ir1):
            body(src_hbm, row2_hbm, col2_hbm, None, out_hbm,
                 acc, bufa, bufb, c0, c1, r0, r1, ga, gb, sa, sb, ic0, ic1,
                 ir0, ir1)
    else:
        @kern
        def k(src_hbm, row2_hbm, col2_hbm, init_hbm, out_hbm,
              acc, bufa, bufb, c0, c1, r0, r1, ga, gb, sa, sb, ic0, ic1,
              ir0, ir1):
            body(src_hbm, row2_hbm, col2_hbm, init_hbm, out_hbm,
                 acc, bufa, bufb, c0, c1, r0, r1, ga, gb, sa, sb, ic0, ic1,
                 ir0, ir1)
    return k


def _tc_init(xp, Wn, bn, W0, b0, dinv2):
    NPAD, DF = xp.shape
    H = Wn.shape[1]
    BR = 1024

    def body(x_ref, wn_ref, bn_ref, w0_ref, b0_ref, dv_ref, xh_ref, hs_ref):
        xh = jnp.maximum(
            jnp.dot(x_ref[...], wn_ref[...],
                    preferred_element_type=jnp.float32,
                    precision=lax.Precision.DEFAULT) + bn_ref[...], 0.0)
        xh_ref[...] = xh
        hs_ref[...] = dv_ref[...] * (
            jnp.dot(xh, w0_ref[...], preferred_element_type=jnp.float32,
                    precision=lax.Precision.DEFAULT) + b0_ref[...])

    return pl.pallas_call(
        body,
        grid=(NPAD // BR,),
        in_specs=[
            pl.BlockSpec((BR, DF), lambda i: (i, 0)),
            pl.BlockSpec((DF, H), lambda i: (0, 0)),
            pl.BlockSpec((1, H), lambda i: (0, 0)),
            pl.BlockSpec((H, H), lambda i: (0, 0)),
            pl.BlockSpec((1, H), lambda i: (0, 0)),
            pl.BlockSpec((BR, 1), lambda i: (i, 0)),
        ],
        out_specs=[pl.BlockSpec((BR, H), lambda i: (i, 0))] * 2,
        out_shape=[jax.ShapeDtypeStruct((NPAD, H), jnp.float32)] * 2,
    )(xp, Wn, bn, W0, b0, dinv2)


def _tc_edge(ea, We, be, dinv_row2, EPAD, BR):
    E, DE = ea.shape
    H = We.shape[1]
    assert E % BR == 0

    def body(ea_ref, we_ref, be_ref, dr_ref, out_ref):
        v = jnp.maximum(
            jnp.dot(ea_ref[...], we_ref[...],
                    preferred_element_type=jnp.float32,
                    precision=lax.Precision.DEFAULT) + be_ref[...], 0.0)
        out_ref[...] = dr_ref[...] * v

    return pl.pallas_call(
        body,
        grid=(E // BR,),
        in_specs=[
            pl.BlockSpec((BR, DE), lambda i: (i, 0)),
            pl.BlockSpec((DE, H), lambda i: (0, 0)),
            pl.BlockSpec((1, H), lambda i: (0, 0)),
            pl.BlockSpec((BR, 1), lambda i: (i, 0)),
        ],
        out_specs=pl.BlockSpec((BR, H), lambda i: (i, 0)),
        out_shape=jax.ShapeDtypeStruct((EPAD, H), jnp.float32),
    )(ea, We, be, dinv_row2)


def _tc_layer(r0, r1, xh, dinv2, W, b):
    NPAD, H = xh.shape
    BR = 1024

    def body(r0_ref, r1_ref, xh_ref, dv_ref, w_ref, b_ref, xh1_ref, hs_ref):
        xh1 = dv_ref[...] * (r0_ref[...] + r1_ref[...]) + xh_ref[...]
        xh1_ref[...] = xh1
        hs_ref[...] = dv_ref[...] * (
            jnp.dot(xh1, w_ref[...], preferred_element_type=jnp.float32,
                    precision=lax.Precision.DEFAULT) + b_ref[...])

    blk = pl.BlockSpec((BR, H), lambda i: (i, 0))
    return pl.pallas_call(
        body,
        grid=(NPAD // BR,),
        in_specs=[blk, blk, blk,
                  pl.BlockSpec((BR, 1), lambda i: (i, 0)),
                  pl.BlockSpec((H, H), lambda i: (0, 0)),
                  pl.BlockSpec((1, H), lambda i: (0, 0))],
        out_specs=[blk, blk],
        out_shape=[jax.ShapeDtypeStruct((NPAD, H), jnp.float32)] * 2,
    )(r0, r1, xh, dinv2, W, b)


def _tc_final(r0, r1, xh, dinv2, batch2, Wf0, bf0, Wf1, bf1):
    NPAD, H = xh.shape
    BR = 1024
    grid = NPAD // BR

    def body(r0_ref, r1_ref, xh_ref, dv_ref, b_ref,
             wf0_ref, bf0_ref, wf1_ref, bf1_ref, out_ref, sums, counts):
        i = pl.program_id(0)

        @pl.when(i == 0)
        def _():
            sums[...] = jnp.zeros_like(sums)
            counts[...] = jnp.zeros_like(counts)

        xh3 = dv_ref[...] * (r0_ref[...] + r1_ref[...]) + xh_ref[...]
        xh3 = jnp.where(b_ref[...] == _G, 0.0, xh3)
        gidx = lax.broadcasted_iota(jnp.int32, (1, _G), 1)
        onehot = (b_ref[...] == gidx).astype(jnp.float32)
        sums[...] += lax.dot_general(
            onehot, xh3, (((0,), (0,)), ((), ())),
            preferred_element_type=jnp.float32,
            precision=lax.Precision.HIGHEST)
        counts[...] += lax.dot_general(
            onehot, jnp.ones((onehot.shape[0], 1), jnp.float32),
            (((0,), (0,)), ((), ())), preferred_element_type=jnp.float32,
            precision=lax.Precision.DEFAULT)

        @pl.when(i == grid - 1)
        def _():
            pooled = sums[...] / jnp.maximum(counts[...], 1.0)
            hidden = jnp.maximum(
                jnp.dot(pooled, wf0_ref[...],
                        preferred_element_type=jnp.float32,
                        precision=lax.Precision.DEFAULT) + bf0_ref[...], 0.0)
            out_ref[...] = jnp.dot(
                hidden, wf1_ref[...], preferred_element_type=jnp.float32,
                precision=lax.Precision.DEFAULT) + bf1_ref[...]

    blk = pl.BlockSpec((BR, H), lambda i: (i, 0))
    return pl.pallas_call(
        body,
        grid=(grid,),
        in_specs=[blk, blk, blk,
                  pl.BlockSpec((BR, 1), lambda i: (i, 0)),
                  pl.BlockSpec((BR, 1), lambda i: (i, 0)),
                  pl.BlockSpec((H, H), lambda i: (0, 0)),
                  pl.BlockSpec((1, H), lambda i: (0, 0)),
                  pl.BlockSpec((H, 1), lambda i: (0, 0)),
                  pl.BlockSpec((1, 1), lambda i: (0, 0))],
        out_specs=pl.BlockSpec((_G, 1), lambda i: (0, 0)),
        out_shape=jax.ShapeDtypeStruct((_G, 1), jnp.float32),
        scratch_shapes=[pltpu.VMEM((_G, H), jnp.float32),
                        pltpu.VMEM((_G, 1), jnp.float32)],
    )(r0, r1, xh, dinv2, batch2, Wf0, bf0, Wf1, bf1)


def kernel(x, edge_index, edge_attr, batch, parity_atoms,
           W_node, b_node, W_edge, b_edge,
           W_conv0, b_conv0, W_conv1, b_conv1, W_conv2, b_conv2,
           W_ffn0, b_ffn0, W_ffn1, b_ffn1):
    N, DF = x.shape
    E = edge_index.shape[1]
    H = W_node.shape[1]
    QN = _NS * _C
    NPAD = ((N + 1 + QN - 1) // QN) * QN
    EQ = _C * _NW * 2 * _BLK
    EPAD = ((E + EQ - 1) // EQ) * EQ

    padidx = (jnp.arange(EPAD - E, dtype=jnp.int32) % (NPAD - N)) + N
    row = jnp.concatenate([edge_index[0], padidx])
    col = jnp.concatenate([edge_index[1], padidx])
    nwch = EPAD // (_C * _NW)
    nsch = EPAD // (_C * _NS)
    col_w2 = col.reshape(_NW * nwch, _C)
    row_w2 = row.reshape(_NW * nwch, _C)
    row_w3 = row.reshape(_NW, nwch, _C)
    col_s3 = col.reshape(_NS, nsch, _C)

    xp = jnp.pad(x, ((0, NPAD - N), (0, 0)))
    batchp = jnp.pad(batch, (0, NPAD - N), constant_values=_G)[:, None]
    zeros_n = jnp.zeros((NPAD,), jnp.float32)

    deg_k = _make_deg_dinv_kernel(NPAD, EPAD)
    gath_k = _make_edge_pass_kernel(NPAD, H, EPAD, gather=True)
    lin_k = _make_edge_pass_kernel(NPAD, H, EPAD, gather=False,
                                   zero_init=True)

    dinv, dinv_row = deg_k(col_s3, row_w3, zeros_n)
    dinv2 = dinv[:, None]

    xh, hs = _tc_init(xp, W_node, b_node.reshape(1, H),
                      W_conv0, b_conv0.reshape(1, H), dinv2)
    ea2 = _tc_edge(edge_attr, W_edge, b_edge.reshape(1, H),
                   dinv_row[:E, None], EPAD, 2560)

    S = lin_k(ea2, row_w2, col_w2)

    for Wl, bl in ((W_conv1, b_conv1), (W_conv2, b_conv2)):
        r = gath_k(hs, row_w2, col_w2, S)
        xh, hs = _tc_layer(r[:NPAD], r[NPAD:], xh, dinv2, Wl,
                           bl.reshape(1, H))

    r = gath_k(hs, row_w2, col_w2, S)
    out = _tc_final(r[:NPAD], r[NPAD:], xh, dinv2, batchp,
                    W_ffn0, b_ffn0.reshape(1, H),
                    W_ffn1, b_ffn1.reshape(1, 1))
    return out


# speedup vs baseline: 1.8889x; 1.0184x over previous
import functools

import jax
import jax.numpy as jnp
from jax import lax
from jax.experimental import pallas as pl
from jax.experimental.pallas import tpu as pltpu
from jax.experimental.pallas import tpu_sc as plsc

_NC = 2
_NS = 16
_L = 16
_NW = _NC * _NS
_G = 64
_C = 128


def _rsqrt_newton(x):
    xi = lax.bitcast_convert_type(x, jnp.int32)
    yi = jnp.int32(0x5F3759DF) - (xi >> 1)
    y = lax.bitcast_convert_type(yi, jnp.float32)
    for _ in range(3):
        y = y * (1.5 - 0.5 * x * y * y)
    return y


def _make_deg_dinv_kernel(NPAD, EPAD):
    assert EPAD % (_C * _NW) == 0
    nsch = EPAD // (_C * _NS)
    nwch = EPAD // (_C * _NW)
    nps = NPAD // _NS
    WIN = 4
    mesh = plsc.VectorSubcoreMesh(core_axis_name="c", subcore_axis_name="s")

    @functools.partial(
        pl.kernel, mesh=mesh,
        compiler_params=pltpu.CompilerParams(needs_layout_passes=False),
        out_type=(jax.ShapeDtypeStruct((NPAD,), jnp.float32),
                  jax.ShapeDtypeStruct((EPAD,), jnp.float32)),
        scratch_types=[
            pltpu.VMEM_SHARED((NPAD,), jnp.float32),
            pltpu.VMEM((nsch, _C), jnp.int32),
            pltpu.VMEM((nwch, _C), jnp.int32),
            pltpu.VMEM((_C,), jnp.float32),
            pltpu.VMEM((NPAD,), jnp.float32),
            pltpu.VMEM((_C,), jnp.float32),
            pltpu.VMEM((_C,), jnp.float32),
            pltpu.SemaphoreType.DMA,
            pltpu.SemaphoreType.DMA,
            pltpu.SemaphoreType.DMA,
        ],
    )
    def k(col3_hbm, row3_hbm, zero_hbm, dinv_hbm, dinvrow_hbm,
          deg_sh, col_v, row_v, ones_v, dinv_v, oa_v, ob_v, ws, sa, sb):
        cid = lax.axis_index("c")
        sid = lax.axis_index("s")
        wid = cid * _NS + sid

        def fill_ones(i, _):
            ones_v[pl.ds(i * _L, _L)] = jnp.full((_L,), 1.0, jnp.float32)
            return 0
        lax.fori_loop(0, _C // _L, fill_ones, 0)

        pltpu.sync_copy(col3_hbm.at[sid], col_v)
        pltpu.sync_copy(row3_hbm.at[wid], row_v)
        pltpu.sync_copy(zero_hbm.at[pl.ds(sid * nps, nps)],
                        deg_sh.at[pl.ds(sid * nps, nps)])
        plsc.subcore_barrier()

        def chunk1(i, _):
            pltpu.async_copy(ones_v, deg_sh.at[col_v.at[i]], ws, add=True)

            @pl.when(i >= WIN)
            def _():
                pltpu.make_async_copy(
                    ones_v, deg_sh.at[col_v.at[i]], ws).wait()
            return 0
        lax.fori_loop(0, nsch, chunk1, 0)
        for i in range(min(WIN, nsch)):
            pltpu.make_async_copy(ones_v, deg_sh.at[col_v.at[i]], ws).wait()
        plsc.subcore_barrier()

        pltpu.sync_copy(deg_sh.at[pl.ds(sid * nps, nps)],
                        dinv_v.at[pl.ds(sid * nps, nps)])

        def conv(i, _):
            o = sid * nps + i * _L
            d = dinv_v[pl.ds(o, _L)]
            r = _rsqrt_newton(jnp.maximum(d, 1.0))
            dinv_v[pl.ds(o, _L)] = jnp.where(d > 0, r, 0.0)
            return 0
        lax.fori_loop(0, nps // _L, conv, 0)
        pltpu.sync_copy(dinv_v.at[pl.ds(sid * nps, nps)],
                        deg_sh.at[pl.ds(sid * nps, nps)])
        plsc.subcore_barrier()
        pltpu.sync_copy(deg_sh, dinv_v)

        @pl.when(cid == 0)
        def _():
            pltpu.sync_copy(dinv_v.at[pl.ds(sid * nps, nps)],
                            dinv_hbm.at[pl.ds(sid * nps, nps)])

        base_w = wid * nwch * _C

        def gath(i, obuf):
            for j in range(_C // _L):
                ids = row_v[i, pl.ds(j * _L, _L)]
                obuf[pl.ds(j * _L, _L)] = plsc.load_gather(dinv_v, [ids])

        def store(i, obuf, sem):
            pltpu.async_copy(
                obuf, dinvrow_hbm.at[pl.ds(base_w + i * _C, _C)], sem)

        def swait(i, obuf, sem):
            pltpu.make_async_copy(
                obuf, dinvrow_hbm.at[pl.ds(base_w + i * _C, _C)], sem).wait()

        obufs = ((oa_v, sa), (ob_v, sb))
        npair = nwch // 2

        def chunk3(p, _):
            for b, (obuf, sem) in enumerate(obufs):
                i = 2 * p + b

                @pl.when(i >= 2)
                def _():
                    swait(i - 2, obuf, sem)
                gath(i, obuf)
                store(i, obuf, sem)
            return 0
        lax.fori_loop(0, npair, chunk3, 0)
        if nwch % 2:
            i = nwch - 1
            if i >= 2:
                swait(i - 2, oa_v, sa)
            gath(i, oa_v)
            store(i, oa_v, sa)
            swait(nwch - 1, oa_v, sa)
            if nwch >= 2:
                swait(nwch - 2, ob_v, sb)
        else:
            if nwch >= 2:
                swait(nwch - 2, oa_v, sa)
            if nwch >= 1:
                swait(nwch - 1, ob_v, sb)

    return k


_BLK = 8


def _make_edge_pass_kernel(NPAD, H, EPAD, gather, zero_init=False):
    assert EPAD % (_C * _NW * 2 * _BLK) == 0 and NPAD % (_NS * _C) == 0
    nwch = EPAD // (_C * _NW)
    nblk = nwch // _BLK
    npairs = nblk // 2
    nps = NPAD // _NS
    mesh = plsc.VectorSubcoreMesh(core_axis_name="c", subcore_axis_name="s")

    scratch = [
        pltpu.VMEM_SHARED((NPAD, H), jnp.float32),
        pltpu.VMEM((_C, H), jnp.float32),
        pltpu.VMEM((_C, H), jnp.float32),
        pltpu.VMEM((_BLK, _C), jnp.int32),
        pltpu.VMEM((_BLK, _C), jnp.int32),
        pltpu.VMEM((_BLK, _C), jnp.int32),
        pltpu.VMEM((_BLK, _C), jnp.int32),
        pltpu.SemaphoreType.DMA,
        pltpu.SemaphoreType.DMA,
        pltpu.SemaphoreType.DMA,
        pltpu.SemaphoreType.DMA,
        pltpu.SemaphoreType.DMA,
        pltpu.SemaphoreType.DMA,
        pltpu.SemaphoreType.DMA,
        pltpu.SemaphoreType.DMA,
    ]

    def body(src_hbm, row2_hbm, col2_hbm, init_hbm, out_hbm,
             acc, bufa, bufb, c0, c1, r0, r1, ga, gb, sa, sb, ic0, ic1,
             ir0, ir1):
        cid = lax.axis_index("c")
        sid = lax.axis_index("s")
        wid = cid * _NS + sid
        base_c = wid * nwch

        def idx_copy(b, cset, rset, csem, rsem):
            src = pl.ds((base_c + b * _BLK), _BLK)
            pltpu.async_copy(col2_hbm.at[src, :], cset, csem)
            if gather:
                pltpu.async_copy(row2_hbm.at[src, :], rset, rsem)

        def idx_wait(b, cset, rset, csem, rsem):
            src = pl.ds((base_c + b * _BLK), _BLK)
            pltpu.make_async_copy(col2_hbm.at[src, :], cset, csem).wait()
            if gather:
                pltpu.make_async_copy(row2_hbm.at[src, :], rset, rsem).wait()

        def gstart(b, j, rset, buf, gsem):
            if gather:
                pltpu.async_copy(src_hbm.at[rset.at[j]], buf, gsem)
            else:
                o = (base_c + b * _BLK + j) * _C
                pltpu.async_copy(src_hbm.at[pl.ds(o, _C), :], buf, gsem)

        def gwait(b, j, rset, buf, gsem):
            if gather:
                pltpu.make_async_copy(src_hbm.at[rset.at[j]], buf,
                                      gsem).wait()
            else:
                o = (base_c + b * _BLK + j) * _C
                pltpu.make_async_copy(src_hbm.at[pl.ds(o, _C), :], buf,
                                      gsem).wait()

        def scat(j, cset, buf, ssem):
            pltpu.async_copy(buf, acc.at[cset.at[j]], ssem, add=True)
            pltpu.make_async_copy(buf, acc.at[cset.at[j]], ssem).wait()

        if zero_init:
            def zrow(rr, _):
                for cc in range(H // _L):
                    bufa[rr, pl.ds(cc * _L, _L)] = jnp.zeros((_L,),
                                                             jnp.float32)
                return 0
            lax.fori_loop(0, _C, zrow, 0)
            for t in range(nps // _C):
                pltpu.async_copy(
                    bufa, acc.at[pl.ds(sid * nps + t * _C, _C), :], ga)
            idx_copy(0, c0, r0, ic0, ir0)
            idx_copy(1, c1, r1, ic1, ir1)
            for t in range(nps // _C):
                pltpu.make_async_copy(
                    bufa, acc.at[pl.ds(sid * nps + t * _C, _C), :],
                    ga).wait()
        else:
            pltpu.sync_copy(
                init_hbm.at[pl.ds(cid * NPAD + sid * nps, nps), :],
                acc.at[pl.ds(sid * nps, nps), :])
            idx_copy(0, c0, r0, ic0, ir0)
            idx_copy(1, c1, r1, ic1, ir1)
        idx_wait(0, c0, r0, ic0, ir0)

        bufs = ((bufa, ga, sa), (bufb, gb, sb))
        gstart(0, 0, r0, bufa, ga)
        gstart(0, 1, r0, bufb, gb)
        plsc.subcore_barrier()

        def block(b, cset, rset, csem, rsem, ncset, nrset, ncsem, nrsem,
                  more):
            for j in range(_BLK):
                buf, gsem, ssem = bufs[j % 2]
                gwait(b, j, rset, buf, gsem)
                scat(j, cset, buf, ssem)
                if j < _BLK - 2:
                    gstart(b, j + 2, rset, buf, gsem)
                else:
                    if j == _BLK - 2:
                        if more is None:
                            idx_wait(b + 1, ncset, nrset, ncsem, nrsem)
                        else:
                            @pl.when(more)
                            def _():
                                idx_wait(b + 1, ncset, nrset, ncsem, nrsem)

                    if more is None:
                        gstart(b + 1, j + 2 - _BLK, nrset, buf, gsem)
                    else:
                        @pl.when(more)
                        def _():
                            gstart(b + 1, j + 2 - _BLK, nrset, buf, gsem)

        def pair(bp, _):
            bA = 2 * bp
            more = bp + 1 < npairs
            block(bA, c0, r0, ic0, ir0, c1, r1, ic1, ir1, None)

            @pl.when(more)
            def _():
                idx_copy(bA + 2, c0, r0, ic0, ir0)
            block(bA + 1, c1, r1, ic1, ir1, c0, r0, ic0, ir0, more)

            @pl.when(more)
            def _():
                idx_copy(bA + 3, c1, r1, ic1, ir1)
            return 0
        lax.fori_loop(0, npairs, pair, 0)

        plsc.subcore_barrier()
        pltpu.sync_copy(acc.at[pl.ds(sid * nps, nps), :],
                        out_hbm.at[pl.ds(cid * NPAD + sid * nps, nps), :])

    kern = functools.partial(
        pl.kernel, mesh=mesh,
        compiler_params=pltpu.CompilerParams(needs_layout_passes=False),
        out_type=jax.ShapeDtypeStruct((_NC * NPAD, H), jnp.float32),
        scratch_types=scratch,
    )
    if zero_init:
        @kern
        def k(src_hbm, row2_hbm, col2_hbm, out_hbm,
              acc, bufa, bufb, c0, c1, r0, r1, ga, gb, sa, sb, ic0, ic1,
              ir0, ir1):
            body(src_hbm, row2_hbm, col2_hbm, None, out_hbm,
                 acc, bufa, bufb, c0, c1, r0, r1, ga, gb, sa, sb, ic0, ic1,
                 ir0, ir1)
    else:
        @kern
        def k(src_hbm, row2_hbm, col2_hbm, init_hbm, out_hbm,
              acc, bufa, bufb, c0, c1, r0, r1, ga, gb, sa, sb, ic0, ic1,
              ir0, ir1):
            body(src_hbm, row2_hbm, col2_hbm, init_hbm, out_hbm,
                 acc, bufa, bufb, c0, c1, r0, r1, ga, gb, sa, sb, ic0, ic1,
                 ir0, ir1)
    return k


def _tc_init(xp, Wn, bn, W0, b0, dinv2):
    NPAD, DF = xp.shape
    H = Wn.shape[1]
    BR = 1024

    def body(x_ref, wn_ref, bn_ref, w0_ref, b0_ref, dv_ref, xh_ref, hs_ref):
        xh = jnp.maximum(
            jnp.dot(x_ref[...], wn_ref[...],
                    preferred_element_type=jnp.float32,
                    precision=lax.Precision.DEFAULT) + bn_ref[...], 0.0)
        xh_ref[...] = xh
        hs_ref[...] = dv_ref[...] * (
            jnp.dot(xh, w0_ref[...], preferred_element_type=jnp.float32,
                    precision=lax.Precision.DEFAULT) + b0_ref[...])

    return pl.pallas_call(
        body,
        grid=(NPAD // BR,),
        in_specs=[
            pl.BlockSpec((BR, DF), lambda i: (i, 0)),
            pl.BlockSpec((DF, H), lambda i: (0, 0)),
            pl.BlockSpec((1, H), lambda i: (0, 0)),
            pl.BlockSpec((H, H), lambda i: (0, 0)),
            pl.BlockSpec((1, H), lambda i: (0, 0)),
            pl.BlockSpec((BR, 1), lambda i: (i, 0)),
        ],
        out_specs=[pl.BlockSpec((BR, H), lambda i: (i, 0))] * 2,
        out_shape=[jax.ShapeDtypeStruct((NPAD, H), jnp.float32)] * 2,
    )(xp, Wn, bn, W0, b0, dinv2)


def _tc_edge(ea, We, be, dinv_row2, EPAD, BR):
    E, DE = ea.shape
    H = We.shape[1]
    assert E % BR == 0

    def body(ea_ref, we_ref, be_ref, dr_ref, out_ref):
        v = jnp.maximum(
            jnp.dot(ea_ref[...], we_ref[...],
                    preferred_element_type=jnp.float32,
                    precision=lax.Precision.DEFAULT) + be_ref[...], 0.0)
        out_ref[...] = dr_ref[...] * v

    return pl.pallas_call(
        body,
        grid=(E // BR,),
        in_specs=[
            pl.BlockSpec((BR, DE), lambda i: (i, 0)),
            pl.BlockSpec((DE, H), lambda i: (0, 0)),
            pl.BlockSpec((1, H), lambda i: (0, 0)),
            pl.BlockSpec((BR, 1), lambda i: (i, 0)),
        ],
        out_specs=pl.BlockSpec((BR, H), lambda i: (i, 0)),
        out_shape=jax.ShapeDtypeStruct((EPAD, H), jnp.float32),
    )(ea, We, be, dinv_row2)


def _tc_layer(r0, r1, xh, dinv2, W, b):
    NPAD, H = xh.shape
    BR = 1024

    def body(r0_ref, r1_ref, xh_ref, dv_ref, w_ref, b_ref, xh1_ref, hs_ref):
        xh1 = dv_ref[...] * (r0_ref[...] + r1_ref[...]) + xh_ref[...]
        xh1_ref[...] = xh1
        hs_ref[...] = dv_ref[...] * (
            jnp.dot(xh1, w_ref[...], preferred_element_type=jnp.float32,
                    precision=lax.Precision.DEFAULT) + b_ref[...])

    blk = pl.BlockSpec((BR, H), lambda i: (i, 0))
    return pl.pallas_call(
        body,
        grid=(NPAD // BR,),
        in_specs=[blk, blk, blk,
                  pl.BlockSpec((BR, 1), lambda i: (i, 0)),
                  pl.BlockSpec((H, H), lambda i: (0, 0)),
                  pl.BlockSpec((1, H), lambda i: (0, 0))],
        out_specs=[blk, blk],
        out_shape=[jax.ShapeDtypeStruct((NPAD, H), jnp.float32)] * 2,
    )(r0, r1, xh, dinv2, W, b)


def _tc_final(r0, r1, xh, dinv2, batch2, Wf0, bf0, Wf1, bf1):
    NPAD, H = xh.shape
    BR = 1024
    grid = NPAD // BR

    def body(r0_ref, r1_ref, xh_ref, dv_ref, b_ref,
             wf0_ref, bf0_ref, wf1_ref, bf1_ref, out_ref, sums, counts):
        i = pl.program_id(0)

        @pl.when(i == 0)
        def _():
            sums[...] = jnp.zeros_like(sums)
            counts[...] = jnp.zeros_like(counts)

        xh3 = dv_ref[...] * (r0_ref[...] + r1_ref[...]) + xh_ref[...]
        xh3 = jnp.where(b_ref[...] == _G, 0.0, xh3)
        gidx = lax.broadcasted_iota(jnp.int32, (1, _G), 1)
        onehot = (b_ref[...] == gidx).astype(jnp.float32)
        sums[...] += lax.dot_general(
            onehot, xh3, (((0,), (0,)), ((), ())),
            preferred_element_type=jnp.float32,
            precision=lax.Precision.HIGHEST)
        counts[...] += lax.dot_general(
            onehot, jnp.ones((onehot.shape[0], 1), jnp.float32),
            (((0,), (0,)), ((), ())), preferred_element_type=jnp.float32,
            precision=lax.Precision.DEFAULT)

        @pl.when(i == grid - 1)
        def _():
            pooled = sums[...] / jnp.maximum(counts[...], 1.0)
            hidden = jnp.maximum(
                jnp.dot(pooled, wf0_ref[...],
                        preferred_element_type=jnp.float32,
                        precision=lax.Precision.DEFAULT) + bf0_ref[...], 0.0)
            out_ref[...] = jnp.dot(
                hidden, wf1_ref[...], preferred_element_type=jnp.float32,
                precision=lax.Precision.DEFAULT) + bf1_ref[...]

    blk = pl.BlockSpec((BR, H), lambda i: (i, 0))
    return pl.pallas_call(
        body,
        grid=(grid,),
        in_specs=[blk, blk, blk,
                  pl.BlockSpec((BR, 1), lambda i: (i, 0)),
                  pl.BlockSpec((BR, 1), lambda i: (i, 0)),
                  pl.BlockSpec((H, H), lambda i: (0, 0)),
                  pl.BlockSpec((1, H), lambda i: (0, 0)),
                  pl.BlockSpec((H, 1), lambda i: (0, 0)),
                  pl.BlockSpec((1, 1), lambda i: (0, 0))],
        out_specs=pl.BlockSpec((_G, 1), lambda i: (0, 0)),
        out_shape=jax.ShapeDtypeStruct((_G, 1), jnp.float32),
        scratch_shapes=[pltpu.VMEM((_G, H), jnp.float32),
                        pltpu.VMEM((_G, 1), jnp.float32)],
    )(r0, r1, xh, dinv2, batch2, Wf0, bf0, Wf1, bf1)


def kernel(x, edge_index, edge_attr, batch, parity_atoms,
           W_node, b_node, W_edge, b_edge,
           W_conv0, b_conv0, W_conv1, b_conv1, W_conv2, b_conv2,
           W_ffn0, b_ffn0, W_ffn1, b_ffn1):
    N, DF = x.shape
    E = edge_index.shape[1]
    H = W_node.shape[1]
    QN = _NS * _C
    NPAD = ((N + 1 + QN - 1) // QN) * QN
    EQ = _C * _NW * 2 * _BLK
    EPAD = ((E + EQ - 1) // EQ) * EQ

    padidx = (jnp.arange(EPAD - E, dtype=jnp.int32) % (NPAD - N)) + N
    row = jnp.concatenate([edge_index[0], padidx])
    col = jnp.concatenate([edge_index[1], padidx])
    nwch = EPAD // (_C * _NW)
    nsch = EPAD // (_C * _NS)
    col_w2 = col.reshape(_NW * nwch, _C)
    row_w2 = row.reshape(_NW * nwch, _C)
    row_w3 = row.reshape(_NW, nwch, _C)
    col_s3 = col.reshape(_NS, nsch, _C)

    xp = jnp.pad(x, ((0, NPAD - N), (0, 0)))
    batchp = jnp.pad(batch, (0, NPAD - N), constant_values=_G)[:, None]
    zeros_n = jnp.zeros((NPAD,), jnp.float32)

    deg_k = _make_deg_dinv_kernel(NPAD, EPAD)
    gath_k = _make_edge_pass_kernel(NPAD, H, EPAD, gather=True)
    lin_k = _make_edge_pass_kernel(NPAD, H, EPAD, gather=False)

    dinv, dinv_row = deg_k(col_s3, row_w3, zeros_n)
    dinv2 = dinv[:, None]

    xh, hs = _tc_init(xp, W_node, b_node.reshape(1, H),
                      W_conv0, b_conv0.reshape(1, H), dinv2)
    eap = jnp.pad(edge_attr, ((0, EPAD - E), (0, 0)))
    ea2 = _tc_edge(eap, W_edge, b_edge.reshape(1, H),
                   dinv_row[:, None], EPAD, 2560)

    zeros_2nh = jnp.zeros((_NC * NPAD, H), jnp.float32)
    S = lin_k(ea2, row_w2, col_w2, zeros_2nh)

    for Wl, bl in ((W_conv1, b_conv1), (W_conv2, b_conv2)):
        r = gath_k(hs, row_w2, col_w2, S)
        xh, hs = _tc_layer(r[:NPAD], r[NPAD:], xh, dinv2, Wl,
                           bl.reshape(1, H))

    r = gath_k(hs, row_w2, col_w2, S)
    out = _tc_final(r[:NPAD], r[NPAD:], xh, dinv2, batchp,
                    W_ffn0, b_ffn0.reshape(1, H),
                    W_ffn1, b_ffn1.reshape(1, 1))
    return out
